# baseline (speedup 1.0000x reference)
import functools

import jax
import jax.numpy as jnp
from jax import lax
from jax.experimental import pallas as pl
from jax.experimental.pallas import tpu as pltpu
from jax.experimental.pallas import tpu_sc as plsc

IN_CH = 128
HID = 128
L = 3
E1 = 320000
E2 = 320000
ET = E1 + E2

NC = 2
NS = 16
NW = NC * NS
RPW = ET // NW
WIN = 250
KT = 248
K2 = 200
FG = IN_CH // 16

N_TRI = 200000 + 3 * 300000 + 3 * 300000 + 200000

_mesh = plsc.VectorSubcoreMesh(core_axis_name="c", subcore_axis_name="s")

_f32 = jnp.float32
_i32 = jnp.int32


def _worker_id():
    return lax.axis_index("s") * NC + lax.axis_index("c")


NWIN = ET // WIN
WPW = NWIN // NW


def _segsum_body(d_hbm, i1_hbm, i2_hbm, wstarts_hbm, state_hbm, out_hbm,
                 ws_s, d_s, i1_v, i2_v, r1, r2, win, sem1, sem2):
    w = _worker_id()
    pltpu.sync_copy(wstarts_hbm.at[pl.ds(w * WPW, WPW + 8)],
                    ws_s.at[pl.ds(0, WPW + 8)])

    zeros16 = jnp.zeros((16,), _f32)

    def _zero_one(j, carry):
        win[pl.ds(j * 16, 16)] = zeros16
        return carry

    def win_body(wi, carry):
        base = (w * WPW + wi) * WIN
        lax.fori_loop(0, WIN * FG, _zero_one, 0)
        bnd = ws_s[pl.ds(wi, 16)]
        lo = bnd[0]
        hi = bnd[1]
        lo_al = (lo // 8) * 8
        nch = (hi - lo_al + KT - 1) // KT

        def chunk_body(ci, c2):
            t0 = lo_al + ci * KT
            pltpu.sync_copy(d_hbm.at[pl.ds(t0, KT)], d_s.at[pl.ds(0, KT)])
            pltpu.sync_copy(i1_hbm.at[pl.ds(t0, KT)], i1_v)
            pltpu.sync_copy(i2_hbm.at[pl.ds(t0, KT)], i2_v)
            cp1 = pltpu.async_copy(state_hbm.at[i1_v], r1, sem1)
            cp2 = pltpu.async_copy(state_hbm.at[i2_v], r2, sem2)
            cp1.wait()
            cp2.wait()
            j0 = jnp.maximum(lo - t0, 0)
            j1 = jnp.minimum(KT, hi - t0)

            def tri_body(j, c3):
                dj = d_s[pl.ds(j, 16)][0]
                off = (dj - base) * IN_CH
                for f in range(FG):
                    v = r1[j, pl.ds(f * 16, 16)] * r2[j, pl.ds(f * 16, 16)]
                    plsc.addupdate(win.at[pl.ds(off + f * 16, 16)], v)
                return c3

            lax.fori_loop(j0, j1, tri_body, 0)
            return c2

        lax.fori_loop(0, nch, chunk_body, 0)
        pltpu.sync_copy(win, out_hbm.at[pl.ds(base * IN_CH, WIN * IN_CH)])
        return carry

    lax.fori_loop(0, WPW, win_body, 0)


def _segsum(d_pad, i1_pad, i2_pad, wstarts, state):
    return pl.kernel(
        _segsum_body,
        out_type=jax.ShapeDtypeStruct((ET * IN_CH,), _f32),
        mesh=_mesh,
        scratch_types=[
            pltpu.VMEM((WPW + 24,), _i32),
            pltpu.VMEM((KT + 16,), _i32),
            pltpu.VMEM((KT,), _i32),
            pltpu.VMEM((KT,), _i32),
            pltpu.VMEM((KT, IN_CH), _f32),
            pltpu.VMEM((KT, IN_CH), _f32),
            pltpu.VMEM((WIN * IN_CH,), _f32),
            pltpu.SemaphoreType.DMA,
            pltpu.SemaphoreType.DMA,
        ],
    )(d_pad, i1_pad, i2_pad, wstarts, state)


def _finish_body(state_hbm, agg_hbm, inv_hbm, y_hbm, inv_v, st, ag, ai, sem):
    w = _worker_id()
    r0 = w * RPW

    def chunk_body(ci, carry):
        rb = r0 + ci * K2
        pltpu.sync_copy(inv_hbm.at[pl.ds(rb, K2)], inv_v)
        cp = pltpu.async_copy(agg_hbm.at[inv_v], ai, sem)
        pltpu.sync_copy(state_hbm.at[pl.ds(rb, K2), :], st)
        pltpu.sync_copy(agg_hbm.at[pl.ds(rb, K2), :], ag)
        cp.wait()

        def row_body(j, c2):
            for f in range(FG):
                sl = pl.ds(f * 16, 16)
                st[j, sl] = st[j, sl] + ag[j, sl] + ai[j, sl]
            return c2

        lax.fori_loop(0, K2, row_body, 0)
        pltpu.sync_copy(st, y_hbm.at[pl.ds(rb, K2), :])
        return carry

    lax.fori_loop(0, RPW // K2, chunk_body, 0)


def _finish(state, agg, inv):
    return pl.kernel(
        _finish_body,
        out_type=jax.ShapeDtypeStruct((ET, IN_CH), _f32),
        mesh=_mesh,
        scratch_types=[
            pltpu.VMEM((K2,), _i32),
            pltpu.VMEM((K2, IN_CH), _f32),
            pltpu.VMEM((K2, IN_CH), _f32),
            pltpu.VMEM((K2, IN_CH), _f32),
            pltpu.SemaphoreType.DMA,
        ],
    )(state, agg, inv)


BL = 512
NB_HALF = E1 // BL
_PREC = jax.lax.Precision.HIGHEST


def _mlp_block(x_ref, w1_ref, b1_ref, w2_ref, b2_ref, o_ref):
    x = x_ref[...]
    h = jnp.dot(x, w1_ref[0], preferred_element_type=_f32, precision=_PREC)
    h = jnp.maximum(h + b1_ref[0], 0.0)
    y = jnp.dot(h, w2_ref[0], preferred_element_type=_f32, precision=_PREC)
    o_ref[...] = jnp.maximum(y + b2_ref[0], 0.0)


def _mlp(y, w1s, b1s, w2s, b2s):
    half = lambda i: i // NB_HALF
    return pl.pallas_call(
        _mlp_block,
        out_shape=jax.ShapeDtypeStruct((ET, IN_CH), _f32),
        grid=(2 * NB_HALF,),
        in_specs=[
            pl.BlockSpec((BL, IN_CH), lambda i: (i, 0)),
            pl.BlockSpec((1, IN_CH, HID), lambda i: (half(i), 0, 0)),
            pl.BlockSpec((1, 1, HID), lambda i: (half(i), 0, 0)),
            pl.BlockSpec((1, HID, IN_CH), lambda i: (half(i), 0, 0)),
            pl.BlockSpec((1, 1, IN_CH), lambda i: (half(i), 0, 0)),
        ],
        out_specs=pl.BlockSpec((BL, IN_CH), lambda i: (i, 0)),
    )(y, w1s, b1s, w2s, b2s)


def _final_block(x_ref, w_ref, b_ref, o_ref):
    y = jnp.dot(x_ref[...], w_ref[0], preferred_element_type=_f32,
                precision=_PREC)
    o_ref[...] = y + b_ref[0]


def _final(state, ws, bs):
    half = lambda i: i // NB_HALF
    return pl.pallas_call(
        _final_block,
        out_shape=jax.ShapeDtypeStruct((ET, IN_CH), _f32),
        grid=(2 * NB_HALF,),
        in_specs=[
            pl.BlockSpec((BL, IN_CH), lambda i: (i, 0)),
            pl.BlockSpec((1, IN_CH, IN_CH), lambda i: (half(i), 0, 0)),
            pl.BlockSpec((1, 1, IN_CH), lambda i: (half(i), 0, 0)),
        ],
        out_specs=pl.BlockSpec((BL, IN_CH), lambda i: (i, 0)),
    )(state, ws, bs)


def kernel(edge_attr, edge_attr2, triangle_1_1_1, triangle_1_1_2,
           triangle_1_2_2, triangle_2_2_2, inverse_edge_1, inverse_edge_2,
           mlp1_w1, mlp1_b1, mlp1_w2, mlp1_b2,
           mlp2_w1, mlp2_b1, mlp2_w2, mlp2_b2,
           fin1_w, fin1_b, fin2_w, fin2_b):
    i32 = lambda x: x.astype(_i32)
    t111 = i32(triangle_1_1_1)
    t112 = i32(triangle_1_1_2)
    t122 = i32(triangle_1_2_2)
    t222 = i32(triangle_2_2_2)

    a, b, c = t111[0], t111[1], t111[2]
    pieces = [(a, b, c)]
    a, b, c = t112[0], t112[1], t112[2]
    pieces += [(a, b, c + E1), (b, a, c + E1), (c + E1, a, b)]
    a, b, c = t122[0], t122[1], t122[2]
    pieces += [(a, b + E1, c + E1), (b + E1, a, c + E1), (c + E1, a, b + E1)]
    a, b, c = t222[0], t222[1], t222[2]
    pieces += [(a + E1, b + E1, c + E1)]

    d = jnp.concatenate([p[0] for p in pieces])
    s1 = jnp.concatenate([p[1] for p in pieces])
    s2 = jnp.concatenate([p[2] for p in pieces])

    d_s, s1_s, s2_s = lax.sort([d, s1, s2], num_keys=1)
    bounds = (jnp.arange(NWIN + 1, dtype=_i32) * WIN)
    wstarts = jnp.searchsorted(d_s, bounds).astype(_i32)

    pad = ((0, KT),)
    d_p = jnp.pad(d_s, pad)
    s1_p = jnp.pad(s1_s, pad)
    s2_p = jnp.pad(s2_s, pad)
    wstarts = jnp.pad(wstarts, ((0, NW * WPW + 8 - (NWIN + 1)),))

    inv = jnp.concatenate([i32(inverse_edge_1), i32(inverse_edge_2) + E1])
    state = jnp.concatenate([edge_attr, edge_attr2], axis=0)

    for i in range(L):
        agg = _segsum(d_p, s1_p, s2_p, wstarts, state).reshape(ET, IN_CH)
        y = _finish(state, agg, inv)
        w1s = jnp.stack([mlp1_w1[i], mlp2_w1[i]])
        b1s = jnp.stack([mlp1_b1[i], mlp2_b1[i]]).reshape(2, 1, HID)
        w2s = jnp.stack([mlp1_w2[i], mlp2_w2[i]])
        b2s = jnp.stack([mlp1_b2[i], mlp2_b2[i]]).reshape(2, 1, IN_CH)
        state = _mlp(y, w1s, b1s, w2s, b2s)

    ws = jnp.stack([fin1_w, fin2_w])
    bs = jnp.stack([fin1_b, fin2_b]).reshape(2, 1, IN_CH)
    out = _final(state, ws, bs)
    return out[:E1], out[E1:]


# speedup vs baseline: 1.3777x; 1.3777x over previous
import functools

import jax
import jax.numpy as jnp
from jax import lax
from jax.experimental import pallas as pl
from jax.experimental.pallas import tpu as pltpu
from jax.experimental.pallas import tpu_sc as plsc

IN_CH = 128
HID = 128
L = 3
E1 = 320000
E2 = 320000
ET = E1 + E2

NC = 2
NS = 16
NW = NC * NS
RPW = ET // NW
WIN = 250
KT = 248
K2 = 200
FG = IN_CH // 16

N_TRI = 200000 + 3 * 300000 + 3 * 300000 + 200000

_mesh = plsc.VectorSubcoreMesh(core_axis_name="c", subcore_axis_name="s")

_f32 = jnp.float32
_i32 = jnp.int32


def _worker_id():
    return lax.axis_index("s") * NC + lax.axis_index("c")


NWIN = ET // WIN
WPW = NWIN // NW


def _segsum_body(d_hbm, i1_hbm, i2_hbm, wstarts_hbm, state_hbm, out_hbm,
                 ws_s, d_s, i1_v, i2_v, r1, r2, win, sem1, sem2):
    w = _worker_id()
    pltpu.sync_copy(wstarts_hbm.at[pl.ds(w * WPW, WPW + 8)],
                    ws_s.at[pl.ds(0, WPW + 8)])

    zeros16 = jnp.zeros((16,), _f32)

    def _zero_one(j, carry):
        for u in range(8):
            win[pl.ds(j * 128 + u * 16, 16)] = zeros16
        return carry

    def win_body(wi, carry):
        base = (w * WPW + wi) * WIN
        lax.fori_loop(0, WIN * FG // 8, _zero_one, 0)
        bnd = ws_s[pl.ds(wi, 16)]
        lo = bnd[0]
        hi = bnd[1]
        lo_al = (lo // 8) * 8
        nch = (hi - lo_al + KT - 1) // KT

        def chunk_body(ci, c2):
            t0 = lo_al + ci * KT
            pltpu.sync_copy(d_hbm.at[pl.ds(t0, KT)], d_s.at[pl.ds(0, KT)])
            pltpu.sync_copy(i1_hbm.at[pl.ds(t0, KT)], i1_v)
            pltpu.sync_copy(i2_hbm.at[pl.ds(t0, KT)], i2_v)
            cp1 = pltpu.async_copy(state_hbm.at[i1_v], r1, sem1)
            cp2 = pltpu.async_copy(state_hbm.at[i2_v], r2, sem2)
            cp1.wait()
            cp2.wait()
            j0 = jnp.maximum(lo - t0, 0)
            j1 = jnp.minimum(KT, hi - t0)

            def tri_body(j, c3):
                dj = d_s[pl.ds(j, 16)][0]
                off = (dj - base) * IN_CH
                va = [r1[j, pl.ds(f * 16, 16)] for f in range(FG)]
                vb = [r2[j, pl.ds(f * 16, 16)] for f in range(FG)]
                for f in range(FG):
                    plsc.addupdate(win.at[pl.ds(off + f * 16, 16)],
                                   va[f] * vb[f])
                return c3

            lax.fori_loop(j0, j1, tri_body, 0)
            return c2

        lax.fori_loop(0, nch, chunk_body, 0)
        pltpu.sync_copy(win, out_hbm.at[pl.ds(base * IN_CH, WIN * IN_CH)])
        return carry

    lax.fori_loop(0, WPW, win_body, 0)


def _segsum(d_pad, i1_pad, i2_pad, wstarts, state):
    return pl.kernel(
        _segsum_body,
        out_type=jax.ShapeDtypeStruct((ET * IN_CH,), _f32),
        mesh=_mesh,
        scratch_types=[
            pltpu.VMEM((WPW + 24,), _i32),
            pltpu.VMEM((KT + 16,), _i32),
            pltpu.VMEM((KT,), _i32),
            pltpu.VMEM((KT,), _i32),
            pltpu.VMEM((KT, IN_CH), _f32),
            pltpu.VMEM((KT, IN_CH), _f32),
            pltpu.VMEM((WIN * IN_CH,), _f32),
            pltpu.SemaphoreType.DMA,
            pltpu.SemaphoreType.DMA,
        ],
    )(d_pad, i1_pad, i2_pad, wstarts, state)


def _finish_body(state_hbm, agg_hbm, inv_hbm, y_hbm, inv_v, st, ag, ai, sem):
    w = _worker_id()
    r0 = w * RPW

    def chunk_body(ci, carry):
        rb = r0 + ci * K2
        pltpu.sync_copy(inv_hbm.at[pl.ds(rb, K2)], inv_v)
        cp = pltpu.async_copy(agg_hbm.at[inv_v], ai, sem)
        pltpu.sync_copy(state_hbm.at[pl.ds(rb, K2), :], st)
        pltpu.sync_copy(agg_hbm.at[pl.ds(rb, K2), :], ag)
        cp.wait()

        def row_body(j, c2):
            for f in range(FG):
                sl = pl.ds(f * 16, 16)
                st[j, sl] = st[j, sl] + ag[j, sl] + ai[j, sl]
            return c2

        lax.fori_loop(0, K2, row_body, 0)
        pltpu.sync_copy(st, y_hbm.at[pl.ds(rb, K2), :])
        return carry

    lax.fori_loop(0, RPW // K2, chunk_body, 0)


def _finish(state, agg, inv):
    return pl.kernel(
        _finish_body,
        out_type=jax.ShapeDtypeStruct((ET, IN_CH), _f32),
        mesh=_mesh,
        scratch_types=[
            pltpu.VMEM((K2,), _i32),
            pltpu.VMEM((K2, IN_CH), _f32),
            pltpu.VMEM((K2, IN_CH), _f32),
            pltpu.VMEM((K2, IN_CH), _f32),
            pltpu.SemaphoreType.DMA,
        ],
    )(state, agg, inv)


BL = 512
NB_HALF = E1 // BL
_PREC = jax.lax.Precision.HIGHEST


def _mlp_block(x_ref, w1_ref, b1_ref, w2_ref, b2_ref, o_ref):
    x = x_ref[...]
    h = jnp.dot(x, w1_ref[0], preferred_element_type=_f32, precision=_PREC)
    h = jnp.maximum(h + b1_ref[0], 0.0)
    y = jnp.dot(h, w2_ref[0], preferred_element_type=_f32, precision=_PREC)
    o_ref[...] = jnp.maximum(y + b2_ref[0], 0.0)


def _mlp(y, w1s, b1s, w2s, b2s):
    half = lambda i: i // NB_HALF
    return pl.pallas_call(
        _mlp_block,
        out_shape=jax.ShapeDtypeStruct((ET, IN_CH), _f32),
        grid=(2 * NB_HALF,),
        in_specs=[
            pl.BlockSpec((BL, IN_CH), lambda i: (i, 0)),
            pl.BlockSpec((1, IN_CH, HID), lambda i: (half(i), 0, 0)),
            pl.BlockSpec((1, 1, HID), lambda i: (half(i), 0, 0)),
            pl.BlockSpec((1, HID, IN_CH), lambda i: (half(i), 0, 0)),
            pl.BlockSpec((1, 1, IN_CH), lambda i: (half(i), 0, 0)),
        ],
        out_specs=pl.BlockSpec((BL, IN_CH), lambda i: (i, 0)),
    )(y, w1s, b1s, w2s, b2s)


def _final_block(x_ref, w_ref, b_ref, o_ref):
    y = jnp.dot(x_ref[...], w_ref[0], preferred_element_type=_f32,
                precision=_PREC)
    o_ref[...] = y + b_ref[0]


def _final(state, ws, bs):
    half = lambda i: i // NB_HALF
    return pl.pallas_call(
        _final_block,
        out_shape=jax.ShapeDtypeStruct((ET, IN_CH), _f32),
        grid=(2 * NB_HALF,),
        in_specs=[
            pl.BlockSpec((BL, IN_CH), lambda i: (i, 0)),
            pl.BlockSpec((1, IN_CH, IN_CH), lambda i: (half(i), 0, 0)),
            pl.BlockSpec((1, 1, IN_CH), lambda i: (half(i), 0, 0)),
        ],
        out_specs=pl.BlockSpec((BL, IN_CH), lambda i: (i, 0)),
    )(state, ws, bs)


def kernel(edge_attr, edge_attr2, triangle_1_1_1, triangle_1_1_2,
           triangle_1_2_2, triangle_2_2_2, inverse_edge_1, inverse_edge_2,
           mlp1_w1, mlp1_b1, mlp1_w2, mlp1_b2,
           mlp2_w1, mlp2_b1, mlp2_w2, mlp2_b2,
           fin1_w, fin1_b, fin2_w, fin2_b):
    i32 = lambda x: x.astype(_i32)
    t111 = i32(triangle_1_1_1)
    t112 = i32(triangle_1_1_2)
    t122 = i32(triangle_1_2_2)
    t222 = i32(triangle_2_2_2)

    a, b, c = t111[0], t111[1], t111[2]
    pieces = [(a, b, c)]
    a, b, c = t112[0], t112[1], t112[2]
    pieces += [(a, b, c + E1), (b, a, c + E1), (c + E1, a, b)]
    a, b, c = t122[0], t122[1], t122[2]
    pieces += [(a, b + E1, c + E1), (b + E1, a, c + E1), (c + E1, a, b + E1)]
    a, b, c = t222[0], t222[1], t222[2]
    pieces += [(a + E1, b + E1, c + E1)]

    d = jnp.concatenate([p[0] for p in pieces])
    s1 = jnp.concatenate([p[1] for p in pieces])
    s2 = jnp.concatenate([p[2] for p in pieces])

    d_s, s1_s, s2_s = lax.sort([d, s1, s2], num_keys=1)
    bounds = (jnp.arange(NWIN + 1, dtype=_i32) * WIN)
    wstarts = jnp.searchsorted(d_s, bounds).astype(_i32)

    pad = ((0, KT),)
    d_p = jnp.pad(d_s, pad)
    s1_p = jnp.pad(s1_s, pad)
    s2_p = jnp.pad(s2_s, pad)
    wstarts = jnp.pad(wstarts, ((0, NW * WPW + 8 - (NWIN + 1)),))

    inv = jnp.concatenate([i32(inverse_edge_1), i32(inverse_edge_2) + E1])
    state = jnp.concatenate([edge_attr, edge_attr2], axis=0)

    for i in range(L):
        agg = _segsum(d_p, s1_p, s2_p, wstarts, state).reshape(ET, IN_CH)
        y = _finish(state, agg, inv)
        w1s = jnp.stack([mlp1_w1[i], mlp2_w1[i]])
        b1s = jnp.stack([mlp1_b1[i], mlp2_b1[i]]).reshape(2, 1, HID)
        w2s = jnp.stack([mlp1_w2[i], mlp2_w2[i]])
        b2s = jnp.stack([mlp1_b2[i], mlp2_b2[i]]).reshape(2, 1, IN_CH)
        state = _mlp(y, w1s, b1s, w2s, b2s)

    ws = jnp.stack([fin1_w, fin2_w])
    bs = jnp.stack([fin1_b, fin2_b]).reshape(2, 1, IN_CH)
    out = _final(state, ws, bs)
    return out[:E1], out[E1:]


# speedup vs baseline: 1.5212x; 1.1042x over previous
import functools

import jax
import jax.numpy as jnp
from jax import lax
from jax.experimental import pallas as pl
from jax.experimental.pallas import tpu as pltpu
from jax.experimental.pallas import tpu_sc as plsc

IN_CH = 128
HID = 128
L = 3
E1 = 320000
E2 = 320000
ET = E1 + E2

NC = 2
NS = 16
NW = NC * NS
RPW = ET // NW
WIN = 250
KT = 144
K2 = 200
FG = IN_CH // 16

N_TRI = 200000 + 3 * 300000 + 3 * 300000 + 200000

_mesh = plsc.VectorSubcoreMesh(core_axis_name="c", subcore_axis_name="s")

_f32 = jnp.float32
_i32 = jnp.int32


def _worker_id():
    return lax.axis_index("s") * NC + lax.axis_index("c")


NWIN = ET // WIN
WPW = NWIN // NW


def _segsum_body(d_hbm, i1_hbm, i2_hbm, wstarts_hbm, state_hbm, out_hbm,
                 ws_s, dA, dB, i1A, i1B, i2A, i2B,
                 r1A, r1B, r2A, r2B, win, semiA, semiB, semgA, semgB):
    w = _worker_id()
    pltpu.sync_copy(wstarts_hbm.at[pl.ds(w * WPW, WPW + 8)],
                    ws_s.at[pl.ds(0, WPW + 8)])

    zeros16 = jnp.zeros((16,), _f32)
    idx_sets = ((dA, i1A, i2A, semiA), (dB, i1B, i2B, semiB))
    r_sets = ((r1A, r2A, semgA), (r1B, r2B, semgB))

    def _zero_one(j, carry):
        for u in range(8):
            win[pl.ds(j * 128 + u * 16, 16)] = zeros16
        return carry

    def _issue_idx(t0, s):
        d_s, i1_v, i2_v, sem = idx_sets[s]
        pltpu.async_copy(d_hbm.at[pl.ds(t0, KT)], d_s.at[pl.ds(0, KT)], sem)
        pltpu.async_copy(i1_hbm.at[pl.ds(t0, KT)], i1_v, sem)
        pltpu.async_copy(i2_hbm.at[pl.ds(t0, KT)], i2_v, sem)

    def _wait_idx(t0, s):
        d_s, i1_v, i2_v, sem = idx_sets[s]
        pltpu.make_async_copy(d_hbm.at[pl.ds(t0, KT)],
                              d_s.at[pl.ds(0, KT)], sem).wait()
        pltpu.make_async_copy(i1_hbm.at[pl.ds(t0, KT)], i1_v, sem).wait()
        pltpu.make_async_copy(i2_hbm.at[pl.ds(t0, KT)], i2_v, sem).wait()

    def _issue_g(s):
        _, i1_v, i2_v, _ = idx_sets[s]
        r1, r2, sem = r_sets[s]
        pltpu.async_copy(state_hbm.at[i1_v], r1, sem)
        pltpu.async_copy(state_hbm.at[i2_v], r2, sem)

    def _wait_g(s):
        _, i1_v, i2_v, _ = idx_sets[s]
        r1, r2, sem = r_sets[s]
        pltpu.make_async_copy(state_hbm.at[i1_v], r1, sem).wait()
        pltpu.make_async_copy(state_hbm.at[i2_v], r2, sem).wait()

    def win_body(wi, carry):
        base = (w * WPW + wi) * WIN
        lax.fori_loop(0, WIN * FG // 8, _zero_one, 0)
        bnd = ws_s[pl.ds(wi, 16)]
        lo = bnd[0]
        hi = bnd[1]
        lo_al = (lo // 8) * 8
        nch = (hi - lo_al + KT - 1) // KT
        npair = (nch + 1) // 2

        _issue_idx(lo_al, 0)
        _wait_idx(lo_al, 0)
        _issue_g(0)
        _issue_idx(lo_al + KT, 1)

        def compute(t0, s):
            d_s = idx_sets[s][0]
            r1, r2, _ = r_sets[s]
            j0 = jnp.maximum(lo - t0, 0)
            j1 = jnp.minimum(KT, hi - t0)

            def tri_body(j, c3):
                dj = d_s[pl.ds(j, 16)][0]
                off = (dj - base) * IN_CH
                va = [r1[j, pl.ds(f * 16, 16)] for f in range(FG)]
                vb = [r2[j, pl.ds(f * 16, 16)] for f in range(FG)]
                for f in range(FG):
                    plsc.addupdate(win.at[pl.ds(off + f * 16, 16)],
                                   va[f] * vb[f])
                return c3

            lax.fori_loop(j0, j1, tri_body, 0)

        def pair_body(pi, carry2):
            for u in (0, 1):
                t0 = lo_al + (2 * pi + u) * KT
                _wait_idx(t0 + KT, 1 - u)
                _issue_g(1 - u)
                _wait_g(u)
                compute(t0, u)
                _issue_idx(t0 + 2 * KT, u)
            return carry2

        lax.fori_loop(0, npair, pair_body, 0)
        t_end = lo_al + 2 * npair * KT
        _wait_g(0)
        _wait_idx(t_end + KT, 1)
        pltpu.sync_copy(win, out_hbm.at[pl.ds(base * IN_CH, WIN * IN_CH)])
        return carry

    lax.fori_loop(0, WPW, win_body, 0)


def _segsum(d_pad, i1_pad, i2_pad, wstarts, state):
    return pl.kernel(
        _segsum_body,
        out_type=jax.ShapeDtypeStruct((ET * IN_CH,), _f32),
        mesh=_mesh,
        scratch_types=[
            pltpu.VMEM((WPW + 24,), _i32),
            pltpu.VMEM((KT + 16,), _i32),
            pltpu.VMEM((KT + 16,), _i32),
            pltpu.VMEM((KT,), _i32),
            pltpu.VMEM((KT,), _i32),
            pltpu.VMEM((KT,), _i32),
            pltpu.VMEM((KT,), _i32),
            pltpu.VMEM((KT, IN_CH), _f32),
            pltpu.VMEM((KT, IN_CH), _f32),
            pltpu.VMEM((KT, IN_CH), _f32),
            pltpu.VMEM((KT, IN_CH), _f32),
            pltpu.VMEM((WIN * IN_CH,), _f32),
            pltpu.SemaphoreType.DMA,
            pltpu.SemaphoreType.DMA,
            pltpu.SemaphoreType.DMA,
            pltpu.SemaphoreType.DMA,
        ],
    )(d_pad, i1_pad, i2_pad, wstarts, state)


def _finish_body(state_hbm, agg_hbm, inv_hbm, y_hbm, inv_v, st, ag, ai, sem):
    w = _worker_id()
    r0 = w * RPW

    def chunk_body(ci, carry):
        rb = r0 + ci * K2
        pltpu.sync_copy(inv_hbm.at[pl.ds(rb, K2)], inv_v)
        cp = pltpu.async_copy(agg_hbm.at[inv_v], ai, sem)
        pltpu.sync_copy(state_hbm.at[pl.ds(rb, K2), :], st)
        pltpu.sync_copy(agg_hbm.at[pl.ds(rb, K2), :], ag)
        cp.wait()

        def row_body(j, c2):
            for f in range(FG):
                sl = pl.ds(f * 16, 16)
                st[j, sl] = st[j, sl] + ag[j, sl] + ai[j, sl]
            return c2

        lax.fori_loop(0, K2, row_body, 0)
        pltpu.sync_copy(st, y_hbm.at[pl.ds(rb, K2), :])
        return carry

    lax.fori_loop(0, RPW // K2, chunk_body, 0)


def _finish(state, agg, inv):
    return pl.kernel(
        _finish_body,
        out_type=jax.ShapeDtypeStruct((ET, IN_CH), _f32),
        mesh=_mesh,
        scratch_types=[
            pltpu.VMEM((K2,), _i32),
            pltpu.VMEM((K2, IN_CH), _f32),
            pltpu.VMEM((K2, IN_CH), _f32),
            pltpu.VMEM((K2, IN_CH), _f32),
            pltpu.SemaphoreType.DMA,
        ],
    )(state, agg, inv)


BL = 512
NB_HALF = E1 // BL
_PREC = jax.lax.Precision.HIGHEST


def _mlp_block(x_ref, w1_ref, b1_ref, w2_ref, b2_ref, o_ref):
    x = x_ref[...]
    h = jnp.dot(x, w1_ref[0], preferred_element_type=_f32, precision=_PREC)
    h = jnp.maximum(h + b1_ref[0], 0.0)
    y = jnp.dot(h, w2_ref[0], preferred_element_type=_f32, precision=_PREC)
    o_ref[...] = jnp.maximum(y + b2_ref[0], 0.0)


def _mlp(y, w1s, b1s, w2s, b2s):
    half = lambda i: i // NB_HALF
    return pl.pallas_call(
        _mlp_block,
        out_shape=jax.ShapeDtypeStruct((ET, IN_CH), _f32),
        grid=(2 * NB_HALF,),
        in_specs=[
            pl.BlockSpec((BL, IN_CH), lambda i: (i, 0)),
            pl.BlockSpec((1, IN_CH, HID), lambda i: (half(i), 0, 0)),
            pl.BlockSpec((1, 1, HID), lambda i: (half(i), 0, 0)),
            pl.BlockSpec((1, HID, IN_CH), lambda i: (half(i), 0, 0)),
            pl.BlockSpec((1, 1, IN_CH), lambda i: (half(i), 0, 0)),
        ],
        out_specs=pl.BlockSpec((BL, IN_CH), lambda i: (i, 0)),
    )(y, w1s, b1s, w2s, b2s)


def _final_block(x_ref, w_ref, b_ref, o_ref):
    y = jnp.dot(x_ref[...], w_ref[0], preferred_element_type=_f32,
                precision=_PREC)
    o_ref[...] = y + b_ref[0]


def _final(state, ws, bs):
    half = lambda i: i // NB_HALF
    return pl.pallas_call(
        _final_block,
        out_shape=jax.ShapeDtypeStruct((ET, IN_CH), _f32),
        grid=(2 * NB_HALF,),
        in_specs=[
            pl.BlockSpec((BL, IN_CH), lambda i: (i, 0)),
            pl.BlockSpec((1, IN_CH, IN_CH), lambda i: (half(i), 0, 0)),
            pl.BlockSpec((1, 1, IN_CH), lambda i: (half(i), 0, 0)),
        ],
        out_specs=pl.BlockSpec((BL, IN_CH), lambda i: (i, 0)),
    )(state, ws, bs)


def kernel(edge_attr, edge_attr2, triangle_1_1_1, triangle_1_1_2,
           triangle_1_2_2, triangle_2_2_2, inverse_edge_1, inverse_edge_2,
           mlp1_w1, mlp1_b1, mlp1_w2, mlp1_b2,
           mlp2_w1, mlp2_b1, mlp2_w2, mlp2_b2,
           fin1_w, fin1_b, fin2_w, fin2_b):
    i32 = lambda x: x.astype(_i32)
    t111 = i32(triangle_1_1_1)
    t112 = i32(triangle_1_1_2)
    t122 = i32(triangle_1_2_2)
    t222 = i32(triangle_2_2_2)

    a, b, c = t111[0], t111[1], t111[2]
    pieces = [(a, b, c)]
    a, b, c = t112[0], t112[1], t112[2]
    pieces += [(a, b, c + E1), (b, a, c + E1), (c + E1, a, b)]
    a, b, c = t122[0], t122[1], t122[2]
    pieces += [(a, b + E1, c + E1), (b + E1, a, c + E1), (c + E1, a, b + E1)]
    a, b, c = t222[0], t222[1], t222[2]
    pieces += [(a + E1, b + E1, c + E1)]

    d = jnp.concatenate([p[0] for p in pieces])
    s1 = jnp.concatenate([p[1] for p in pieces])
    s2 = jnp.concatenate([p[2] for p in pieces])

    d_s, s1_s, s2_s = lax.sort([d, s1, s2], num_keys=1)
    bounds = (jnp.arange(NWIN + 1, dtype=_i32) * WIN)
    wstarts = jnp.searchsorted(d_s, bounds).astype(_i32)

    pad = ((0, 4 * KT),)
    d_p = jnp.pad(d_s, pad)
    s1_p = jnp.pad(s1_s, pad)
    s2_p = jnp.pad(s2_s, pad)
    wstarts = jnp.pad(wstarts, ((0, NW * WPW + 8 - (NWIN + 1)),))

    inv = jnp.concatenate([i32(inverse_edge_1), i32(inverse_edge_2) + E1])
    state = jnp.concatenate([edge_attr, edge_attr2], axis=0)

    for i in range(L):
        agg = _segsum(d_p, s1_p, s2_p, wstarts, state).reshape(ET, IN_CH)
        y = _finish(state, agg, inv)
        w1s = jnp.stack([mlp1_w1[i], mlp2_w1[i]])
        b1s = jnp.stack([mlp1_b1[i], mlp2_b1[i]]).reshape(2, 1, HID)
        w2s = jnp.stack([mlp1_w2[i], mlp2_w2[i]])
        b2s = jnp.stack([mlp1_b2[i], mlp2_b2[i]]).reshape(2, 1, IN_CH)
        state = _mlp(y, w1s, b1s, w2s, b2s)

    ws = jnp.stack([fin1_w, fin2_w])
    bs = jnp.stack([fin1_b, fin2_b]).reshape(2, 1, IN_CH)
    out = _final(state, ws, bs)
    return out[:E1], out[E1:]
